# baseline (speedup 1.0000x reference)
import functools

import jax
import jax.numpy as jnp
from jax import lax
from jax.experimental import pallas as pl
from jax.experimental.pallas import tpu as pltpu
from jax.experimental.pallas import tpu_sc as plsc

N = 10000
E = 320000
D = 128

NC = 2
NS = 16
HALF = N // NC

CHUNK = 80
CPT = 256
EPT = CPT * CHUNK
E_PAD = EPT * NS
STRIPE = 576
R_PAD = STRIPE * NS
EPS = 1e-5


NBUF = 4


def _sc_scatter_body(h_hbm, src_hbm, dst_hbm, zblk_hbm, out_hbm,
                     sidxs, didxs, rows, isems, gsems, ssems, agg):
    c = lax.axis_index("c")
    s = lax.axis_index("s")

    pltpu.sync_copy(zblk_hbm, agg.at[pl.ds(s * STRIPE, STRIPE)])
    plsc.subcore_barrier()

    base = c * E_PAD + s * EPT

    def idx_copy(j, k):
        pltpu.async_copy(src_hbm.at[pl.ds(base + j * CHUNK, CHUNK)],
                         sidxs[k], isems[k])
        pltpu.async_copy(dst_hbm.at[pl.ds(base + j * CHUNK, CHUNK)],
                         didxs[k], isems[k])

    def idx_wait(k):
        pltpu.make_async_copy(src_hbm.at[pl.ds(0, CHUNK)], sidxs[k],
                              isems[k]).wait()
        pltpu.make_async_copy(dst_hbm.at[pl.ds(0, CHUNK)], didxs[k],
                              isems[k]).wait()

    for k in range(NBUF):
        idx_copy(k, k)

    def group(i, carry):
        gats = []
        for k in range(NBUF):
            idx_wait(k)
            gats.append(pltpu.async_copy(h_hbm.at[sidxs[k]], rows[k],
                                         gsems[k]))
        scats = []
        for k in range(NBUF):
            gats[k].wait()
            scats.append(pltpu.async_copy(rows[k], agg.at[didxs[k]],
                                          ssems[k], add=True))
        for k in range(NBUF):
            j = i * NBUF + k + NBUF
            scats[k].wait()

            @pl.when(j < CPT)
            def _():
                idx_copy(j, k)
        return carry

    lax.fori_loop(0, CPT // NBUF, group, jnp.int32(0))
    plsc.subcore_barrier()
    pltpu.sync_copy(agg.at[pl.ds(s * STRIPE, STRIPE)],
                    out_hbm.at[c, pl.ds(s * STRIPE, STRIPE)])


_sc_scatter = functools.partial(
    pl.kernel,
    out_type=jax.ShapeDtypeStruct((NC, R_PAD, D), jnp.float32),
    mesh=plsc.VectorSubcoreMesh(core_axis_name="c", subcore_axis_name="s"),
    scratch_types=[
        [pltpu.VMEM((CHUNK,), jnp.int32) for _ in range(NBUF)],
        [pltpu.VMEM((CHUNK,), jnp.int32) for _ in range(NBUF)],
        [pltpu.VMEM((CHUNK, D), jnp.float32) for _ in range(NBUF)],
        [pltpu.SemaphoreType.DMA for _ in range(NBUF)],
        [pltpu.SemaphoreType.DMA for _ in range(NBUF)],
        [pltpu.SemaphoreType.DMA for _ in range(NBUF)],
        pltpu.VMEM_SHARED((R_PAD, D), jnp.float32),
    ],
)(_sc_scatter_body)


def _mm_body(x_ref, w_ref, o_ref):
    o_ref[...] = jnp.dot(x_ref[...], w_ref[...],
                         preferred_element_type=jnp.float32)


def _matmul(x, w):
    return pl.pallas_call(
        _mm_body,
        grid=(5,),
        in_specs=[pl.BlockSpec((N // 5, D), lambda i: (i, 0)),
                  pl.BlockSpec((D, D), lambda i: (0, 0))],
        out_specs=pl.BlockSpec((N // 5, D), lambda i: (i, 0)),
        out_shape=jax.ShapeDtypeStruct((N, D), jnp.float32),
    )(x, w)


def _bn_core(parts_ref, b_ref, g_ref, bt_ref):
    t = jnp.concatenate([parts_ref[0, :HALF, :], parts_ref[1, :HALF, :]],
                        axis=0) + b_ref[...]
    t = jnp.maximum(t, 0.0)
    mu = jnp.mean(t, axis=0, keepdims=True)
    var = jnp.mean((t - mu) ** 2, axis=0, keepdims=True)
    return (t - mu) * lax.rsqrt(var + EPS) * g_ref[...] + bt_ref[...]


def _bn_mm_body(parts_ref, b_ref, g_ref, bt_ref, w_ref, o_ref):
    y = _bn_core(parts_ref, b_ref, g_ref, bt_ref)
    o_ref[...] = jnp.dot(y, w_ref[...], preferred_element_type=jnp.float32)


def _bn_body(parts_ref, b_ref, g_ref, bt_ref, o_ref):
    o_ref[...] = _bn_core(parts_ref, b_ref, g_ref, bt_ref)


def _bn_mm(parts, b, g, bt, w):
    return pl.pallas_call(
        _bn_mm_body,
        out_shape=jax.ShapeDtypeStruct((N, D), jnp.float32),
    )(parts, b.reshape(1, D), g.reshape(1, D), bt.reshape(1, D), w)


def _bn(parts, b, g, bt):
    return pl.pallas_call(
        _bn_body,
        out_shape=jax.ShapeDtypeStruct((N, D), jnp.float32),
    )(parts, b.reshape(1, D), g.reshape(1, D), bt.reshape(1, D))


def kernel(x, adj, useless, W1, b1, g1, bt1, W2, b2, g2, bt2, W3, b3, g3, bt3):
    pad = E_PAD - E
    src = jnp.concatenate([adj[0], jnp.zeros((pad,), jnp.int32)])
    dst = jnp.concatenate([adj[1], jnp.full((pad,), N, jnp.int32)])
    pos = jnp.arange(E_PAD, dtype=jnp.int32)
    dsts, srcs = [], []
    for c in range(NC):
        dl = dst - c * HALF
        ok = (dl >= 0) & (dl < HALF)
        dsts.append(jnp.where(ok, dl, HALF + (pos & 4095)))
        srcs.append(jnp.where(ok, src, ((pos // CHUNK) * 37) & 1023))
    dst2 = jnp.concatenate(dsts)
    src2 = jnp.concatenate(srcs)
    zblk = jnp.zeros((STRIPE, D), jnp.float32)

    h = _matmul(x, W1)
    parts = _sc_scatter(h, src2, dst2, zblk)
    h = _bn_mm(parts, b1, g1, bt1, W2)
    parts = _sc_scatter(h, src2, dst2, zblk)
    h = _bn_mm(parts, b2, g2, bt2, W3)
    parts = _sc_scatter(h, src2, dst2, zblk)
    return _bn(parts, b3, g3, bt3)


# speedup vs baseline: 2.0847x; 2.0847x over previous
import functools

import jax
import jax.numpy as jnp
from jax import lax
from jax.experimental import pallas as pl
from jax.experimental.pallas import tpu as pltpu
from jax.experimental.pallas import tpu_sc as plsc

N = 10000
E = 320000
D = 128

NC = 2
NS = 16
HALF = N // NC

CHUNK = 80
CPT = 260
EPT = CPT * CHUNK
E_PAD = EPT * NS
STRIPE = 576
R_PAD = STRIPE * NS
EPS = 1e-5


NBUF = 5


def _sc_scatter_body(h_hbm, src_hbm, dst_hbm, zblk_hbm, out_hbm,
                     sidxs, didxs, rows, isems, gsems, ssems, agg):
    c = lax.axis_index("c")
    s = lax.axis_index("s")

    pltpu.sync_copy(zblk_hbm, agg.at[pl.ds(s * STRIPE, STRIPE)])
    plsc.subcore_barrier()

    base = c * E_PAD + s * EPT

    def idx_copy(j, k):
        pltpu.async_copy(src_hbm.at[pl.ds(base + j * CHUNK, CHUNK)],
                         sidxs[k], isems[k])
        pltpu.async_copy(dst_hbm.at[pl.ds(base + j * CHUNK, CHUNK)],
                         didxs[k], isems[k])

    def idx_wait(k):
        pltpu.make_async_copy(src_hbm.at[pl.ds(0, CHUNK)], sidxs[k],
                              isems[k]).wait()
        pltpu.make_async_copy(dst_hbm.at[pl.ds(0, CHUNK)], didxs[k],
                              isems[k]).wait()

    for k in range(NBUF):
        idx_copy(k, k)

    def group(i, carry):
        gats = []
        for k in range(NBUF):
            idx_wait(k)
            gats.append(pltpu.async_copy(h_hbm.at[sidxs[k]], rows[k],
                                         gsems[k]))
        scats = []
        for k in range(NBUF):
            gats[k].wait()
            scats.append(pltpu.async_copy(rows[k], agg.at[didxs[k]],
                                          ssems[k], add=True))
        for k in range(NBUF):
            j = i * NBUF + k + NBUF
            scats[k].wait()

            @pl.when(j < CPT)
            def _():
                idx_copy(j, k)
        return carry

    lax.fori_loop(0, CPT // NBUF, group, jnp.int32(0))
    plsc.subcore_barrier()
    pltpu.sync_copy(agg.at[pl.ds(s * STRIPE, STRIPE)],
                    out_hbm.at[c, pl.ds(s * STRIPE, STRIPE)])


_sc_scatter = functools.partial(
    pl.kernel,
    out_type=jax.ShapeDtypeStruct((NC, R_PAD, D), jnp.float32),
    mesh=plsc.VectorSubcoreMesh(core_axis_name="c", subcore_axis_name="s"),
    scratch_types=[
        [pltpu.VMEM((CHUNK,), jnp.int32) for _ in range(NBUF)],
        [pltpu.VMEM((CHUNK,), jnp.int32) for _ in range(NBUF)],
        [pltpu.VMEM((CHUNK, D), jnp.float32) for _ in range(NBUF)],
        [pltpu.SemaphoreType.DMA for _ in range(NBUF)],
        [pltpu.SemaphoreType.DMA for _ in range(NBUF)],
        [pltpu.SemaphoreType.DMA for _ in range(NBUF)],
        pltpu.VMEM_SHARED((R_PAD, D), jnp.float32),
    ],
)(_sc_scatter_body)


def _mm_body(x_ref, w_ref, o_ref):
    o_ref[...] = jnp.dot(x_ref[...], w_ref[...],
                         preferred_element_type=jnp.float32)


def _matmul(x, w):
    return pl.pallas_call(
        _mm_body,
        grid=(5,),
        in_specs=[pl.BlockSpec((N // 5, D), lambda i: (i, 0)),
                  pl.BlockSpec((D, D), lambda i: (0, 0))],
        out_specs=pl.BlockSpec((N // 5, D), lambda i: (i, 0)),
        out_shape=jax.ShapeDtypeStruct((N, D), jnp.float32),
    )(x, w)


def _bn_core(parts_ref, b_ref, g_ref, bt_ref):
    t = jnp.concatenate([parts_ref[0, :HALF, :], parts_ref[1, :HALF, :]],
                        axis=0) + b_ref[...]
    t = jnp.maximum(t, 0.0)
    mu = jnp.mean(t, axis=0, keepdims=True)
    var = jnp.mean((t - mu) ** 2, axis=0, keepdims=True)
    return (t - mu) * lax.rsqrt(var + EPS) * g_ref[...] + bt_ref[...]


def _bn_mm_body(parts_ref, b_ref, g_ref, bt_ref, w_ref, o_ref):
    y = _bn_core(parts_ref, b_ref, g_ref, bt_ref)
    o_ref[...] = jnp.dot(y, w_ref[...], preferred_element_type=jnp.float32)


def _bn_body(parts_ref, b_ref, g_ref, bt_ref, o_ref):
    o_ref[...] = _bn_core(parts_ref, b_ref, g_ref, bt_ref)


def _bn_mm(parts, b, g, bt, w):
    return pl.pallas_call(
        _bn_mm_body,
        out_shape=jax.ShapeDtypeStruct((N, D), jnp.float32),
    )(parts, b.reshape(1, D), g.reshape(1, D), bt.reshape(1, D), w)


def _bn(parts, b, g, bt):
    return pl.pallas_call(
        _bn_body,
        out_shape=jax.ShapeDtypeStruct((N, D), jnp.float32),
    )(parts, b.reshape(1, D), g.reshape(1, D), bt.reshape(1, D))


def kernel(x, adj, useless, W1, b1, g1, bt1, W2, b2, g2, bt2, W3, b3, g3, bt3):
    pad = E_PAD - E
    src = jnp.concatenate([adj[0], jnp.zeros((pad,), jnp.int32)])
    dst = jnp.concatenate([adj[1], jnp.full((pad,), N, jnp.int32)])
    pos = jnp.arange(E_PAD, dtype=jnp.int32)
    dsts, srcs = [], []
    for c in range(NC):
        dl = dst - c * HALF
        ok = (dl >= 0) & (dl < HALF)
        dsts.append(jnp.where(ok, dl, HALF + (pos & 4095)))
        srcs.append(jnp.where(ok, src, (pos * 37) & 1023))
    dst2 = jnp.concatenate(dsts)
    src2 = jnp.concatenate(srcs)
    zblk = jnp.zeros((STRIPE, D), jnp.float32)

    h = _matmul(x, W1)
    parts = _sc_scatter(h, src2, dst2, zblk)
    h = _bn_mm(parts, b1, g1, bt1, W2)
    parts = _sc_scatter(h, src2, dst2, zblk)
    h = _bn_mm(parts, b2, g2, bt2, W3)
    parts = _sc_scatter(h, src2, dst2, zblk)
    return _bn(parts, b3, g3, bt3)
